# baseline (speedup 1.0000x reference)
import functools

import jax
import jax.numpy as jnp
from jax import lax
from jax.experimental import pallas as pl
from jax.experimental.pallas import tpu as pltpu
from jax.experimental.pallas import tpu_sc as plsc


def _build_ver(put_idx, max_size):
    info = plsc.get_sparse_core_info()
    nc, ns, lanes = info.num_cores, info.num_subcores, info.num_lanes
    nw = nc * ns
    n_put = put_idx.shape[0]
    vpt = 1 << max(-(-max_size // nw) - 1, 1).bit_length()
    ver_total = vpt * nw
    chunk = 16384
    nchunk = n_put // chunk
    unroll = 4
    assert n_put % chunk == 0 and chunk % (lanes * unroll) == 0

    mesh = plsc.VectorSubcoreMesh(core_axis_name="c", subcore_axis_name="s")

    @functools.partial(
        pl.kernel,
        mesh=mesh,
        out_type=jax.ShapeDtypeStruct((ver_total,), jnp.int32),
        scratch_types=[
            pltpu.VMEM((chunk,), jnp.int32),
            pltpu.VMEM((chunk,), jnp.int32),
            pltpu.VMEM((vpt,), jnp.int32),
            pltpu.SemaphoreType.DMA,
            pltpu.SemaphoreType.DMA,
        ],
        compiler_params=pltpu.CompilerParams(needs_layout_passes=False),
    )
    def ka(put_hbm, ver_hbm, ch0, ch1, verv, sem0, sem1):
        wid = lax.axis_index("s") * nc + lax.axis_index("c")
        lo = wid * vpt
        zero16 = jnp.zeros((lanes,), jnp.int32)
        iota1 = jnp.arange(lanes, dtype=jnp.int32) + 1

        bufs = (ch0, ch1)
        sems = (sem0, sem1)
        copies = [None, None]
        copies[0] = pltpu.async_copy(put_hbm.at[pl.ds(0, chunk)], ch0, sem0)

        def zbody(i, _):
            zb = i * (lanes * 8)
            for u in range(8):
                verv[pl.ds(zb + u * lanes, lanes)] = zero16
            return 0

        lax.fori_loop(0, vpt // (lanes * 8), zbody, 0)

        for c in range(nchunk):
            if c + 1 < nchunk:
                copies[(c + 1) % 2] = pltpu.async_copy(
                    put_hbm.at[pl.ds((c + 1) * chunk, chunk)],
                    bufs[(c + 1) % 2], sems[(c + 1) % 2])
            copies[c % 2].wait()
            ch = bufs[c % 2]

            def vbody(k, _, _c=c, _ch=ch):
                vb = k * (lanes * unroll)
                for u in range(unroll):
                    off = vb + u * lanes
                    idx = _ch[pl.ds(off, lanes)]
                    loc = idx - lo
                    m = loc.astype(jnp.uint32) < jnp.uint32(vpt)
                    locc = loc & (vpt - 1)
                    jv = iota1 + (_c * chunk + off)
                    plsc.store_scatter(verv, [locc], jv, mask=m)
                return 0

            lax.fori_loop(0, chunk // (lanes * unroll), vbody, 0)

        pltpu.sync_copy(verv, ver_hbm.at[pl.ds(lo, vpt)])

    return ka(put_idx)


def _gather_mem_cols(mem128, sample_idx):
    info = plsc.get_sparse_core_info()
    nc, ns, lanes = info.num_cores, info.num_subcores, info.num_lanes
    nw = nc * ns
    n_sample = sample_idx.shape[0]
    spt = n_sample // nw
    chunk = 256
    nchunk = spt // chunk
    assert spt % chunk == 0 and chunk % lanes == 0

    mesh = plsc.VectorSubcoreMesh(core_axis_name="c", subcore_axis_name="s")

    @functools.partial(
        pl.kernel,
        mesh=mesh,
        out_type=tuple(
            jax.ShapeDtypeStruct((n_sample,), jnp.float32) for _ in range(8)),
        scratch_types=[
            pltpu.VMEM((spt,), jnp.int32),
            pltpu.VMEM((spt,), jnp.int32),
            pltpu.VMEM((chunk, 128), jnp.float32),
            pltpu.VMEM((chunk, 128), jnp.float32),
            pltpu.VMEM((8, spt), jnp.float32),
            pltpu.SemaphoreType.DMA,
            pltpu.SemaphoreType.DMA,
        ],
        compiler_params=pltpu.CompilerParams(needs_layout_passes=False),
    )
    def km(mem_hbm, sidx_hbm, o0, o1, o2, o3, o4, o5, o6, o7,
           sidxv, sg, bb0, bb1, colsv, sem0, sem1):
        wid = lax.axis_index("s") * nc + lax.axis_index("c")
        base = wid * spt
        outs = (o0, o1, o2, o3, o4, o5, o6, o7)
        pltpu.sync_copy(sidx_hbm.at[pl.ds(base, spt)], sidxv)

        def gbody(k, _):
            s = sidxv[pl.ds(k * lanes, lanes)]
            sg[pl.ds(k * lanes, lanes)] = s >> 4
            return 0

        lax.fori_loop(0, spt // lanes, gbody, 0)

        iota = jnp.arange(lanes, dtype=jnp.int32)
        bufs = (bb0, bb1)
        sems = (sem0, sem1)
        copies = [None, None]
        copies[0] = pltpu.async_copy(
            mem_hbm.at[sg.at[pl.ds(0, chunk)]], bb0, sem0)
        for c in range(nchunk):
            if c + 1 < nchunk:
                copies[(c + 1) % 2] = pltpu.async_copy(
                    mem_hbm.at[sg.at[pl.ds((c + 1) * chunk, chunk)]],
                    bufs[(c + 1) % 2], sems[(c + 1) % 2])
            copies[c % 2].wait()
            bb = bufs[c % 2]

            def ebody(k, _, _c=c, _bb=bb):
                off = _c * chunk + k * lanes
                sv = sidxv[pl.ds(off, lanes)]
                sub = (sv & 15) * 8
                rvec = iota + k * lanes
                for co in range(8):
                    val = plsc.load_gather(_bb, [rvec, sub + co])
                    colsv[co, pl.ds(off, lanes)] = val
                return 0

            lax.fori_loop(0, chunk // lanes, ebody, 0)

        for co in range(8):
            pltpu.sync_copy(colsv.at[co], outs[co].at[pl.ds(base, spt)])

    return km(mem128, sample_idx)


def _sample(mem_cols, put_val, sample_idx, ver2d):
    info = plsc.get_sparse_core_info()
    nc, ns, lanes = info.num_cores, info.num_subcores, info.num_lanes
    nw = nc * ns
    n_sample = sample_idx.shape[0]
    spt = n_sample // nw
    assert spt % lanes == 0

    mesh = plsc.VectorSubcoreMesh(core_axis_name="c", subcore_axis_name="s")

    @functools.partial(
        pl.kernel,
        mesh=mesh,
        out_type=tuple(
            jax.ShapeDtypeStruct((n_sample,), jnp.float32) for _ in range(8)),
        scratch_types=[
            pltpu.VMEM((spt,), jnp.int32),
            pltpu.VMEM((spt,), jnp.int32),
            pltpu.VMEM((spt, 8), jnp.int32),
            pltpu.VMEM((spt,), jnp.int32),
            pltpu.VMEM((spt,), jnp.int32),
            pltpu.VMEM((8, spt), jnp.float32),
            pltpu.VMEM((spt, 8), jnp.float32),
            pltpu.VMEM((8, spt), jnp.float32),
            pltpu.SemaphoreType.DMA,
            pltpu.SemaphoreType.DMA,
        ],
        compiler_params=pltpu.CompilerParams(
            needs_layout_passes=False, use_tc_tiling_on_sc=False),
    )
    def kb(m0, m1, m2, m3, m4, m5, m6, m7, pval_hbm, sidx_hbm, ver_hbm,
           o0, o1, o2, o3, o4, o5, o6, o7,
           sidxv, sg, vrows, vv, pv, mcolv, pvalv, selv, sem1, sem2):
        wid = lax.axis_index("s") * nc + lax.axis_index("c")
        base = wid * spt
        outs = (o0, o1, o2, o3, o4, o5, o6, o7)
        mins = (m0, m1, m2, m3, m4, m5, m6, m7)
        pltpu.sync_copy(sidx_hbm.at[pl.ds(base, spt)], sidxv)
        for co in range(8):
            pltpu.sync_copy(mins[co].at[pl.ds(base, spt)], mcolv.at[co])

        def gbody(k, _):
            s = sidxv[pl.ds(k * lanes, lanes)]
            sg[pl.ds(k * lanes, lanes)] = s >> 3
            return 0

        lax.fori_loop(0, spt // lanes, gbody, 0)
        cp_ver = pltpu.async_copy(ver_hbm.at[sg], vrows, sem2)
        cp_ver.wait()

        iota = jnp.arange(lanes, dtype=jnp.int32)

        def pbody(k, _):
            s = sidxv[pl.ds(k * lanes, lanes)]
            rvec = iota + k * lanes
            v = plsc.load_gather(vrows, [rvec, s & 7])
            vv[pl.ds(k * lanes, lanes)] = v
            pv[pl.ds(k * lanes, lanes)] = jnp.maximum(v - 1, 0)
            return 0

        lax.fori_loop(0, spt // lanes, pbody, 0)
        cp_pval = pltpu.async_copy(pval_hbm.at[pv], pvalv, sem2)
        cp_pval.wait()

        def sbody(k, _):
            vvv = vv[pl.ds(k * lanes, lanes)]
            m = vvv > 0
            rvec = iota + k * lanes
            for co in range(8):
                cosplat = jnp.full((lanes,), co, jnp.int32)
                mval = mcolv[co, pl.ds(k * lanes, lanes)]
                pval = plsc.load_gather(pvalv, [rvec, cosplat])
                sel = jnp.where(m, pval, mval)
                selv[co, pl.ds(k * lanes, lanes)] = sel
            return 0

        lax.fori_loop(0, spt // lanes, sbody, 0)

        for co in range(8):
            pltpu.sync_copy(selv.at[co], outs[co].at[pl.ds(base, spt)])

    return kb(*mem_cols, put_val, sample_idx, ver2d)


def kernel(mem, put_idx, put_val, sample_idx):
    put_idx = put_idx.astype(jnp.int32)
    sample_idx = sample_idx.astype(jnp.int32)
    ver = _build_ver(put_idx, mem.shape[0])
    ver2d = ver.reshape(-1, 8)
    mem128 = mem.reshape(-1, 128)
    mem_cols = _gather_mem_cols(mem128, sample_idx)
    cols = _sample(mem_cols, put_val, sample_idx, ver2d)
    state = jnp.stack(cols[0:3], axis=1)
    action = cols[3].reshape(-1, 1)
    reward = cols[4].reshape(-1, 1)
    next_state = jnp.stack(cols[5:8], axis=1)
    return (state, action, reward, next_state)


# speedup vs baseline: 1.3046x; 1.3046x over previous
import functools

import jax
import jax.numpy as jnp
from jax import lax
from jax.experimental import pallas as pl
from jax.experimental.pallas import tpu as pltpu
from jax.experimental.pallas import tpu_sc as plsc


def _build_ver(put_idx, max_size):
    info = plsc.get_sparse_core_info()
    nc, ns, lanes = info.num_cores, info.num_subcores, info.num_lanes
    nw = nc * ns
    n_put = put_idx.shape[0]
    vpt = 1 << max(-(-max_size // nw) - 1, 1).bit_length()
    ver_total = vpt * nw
    chunk = 16384
    nchunk = n_put // chunk
    unroll = 4
    assert n_put % chunk == 0 and chunk % (lanes * unroll) == 0

    mesh = plsc.VectorSubcoreMesh(core_axis_name="c", subcore_axis_name="s")

    @functools.partial(
        pl.kernel,
        mesh=mesh,
        out_type=jax.ShapeDtypeStruct((ver_total,), jnp.int32),
        scratch_types=[
            pltpu.VMEM((chunk,), jnp.int32),
            pltpu.VMEM((chunk,), jnp.int32),
            pltpu.VMEM((vpt,), jnp.int32),
            pltpu.SemaphoreType.DMA,
            pltpu.SemaphoreType.DMA,
        ],
        compiler_params=pltpu.CompilerParams(needs_layout_passes=False),
    )
    def ka(put_hbm, ver_hbm, ch0, ch1, verv, sem0, sem1):
        wid = lax.axis_index("s") * nc + lax.axis_index("c")
        lo = wid * vpt
        zero16 = jnp.zeros((lanes,), jnp.int32)
        iota1 = jnp.arange(lanes, dtype=jnp.int32) + 1

        bufs = (ch0, ch1)
        sems = (sem0, sem1)
        copies = [None, None]
        copies[0] = pltpu.async_copy(put_hbm.at[pl.ds(0, chunk)], ch0, sem0)

        def zbody(i, _):
            zb = i * (lanes * 8)
            for u in range(8):
                verv[pl.ds(zb + u * lanes, lanes)] = zero16
            return 0

        lax.fori_loop(0, vpt // (lanes * 8), zbody, 0)

        for c in range(nchunk):
            if c + 1 < nchunk:
                copies[(c + 1) % 2] = pltpu.async_copy(
                    put_hbm.at[pl.ds((c + 1) * chunk, chunk)],
                    bufs[(c + 1) % 2], sems[(c + 1) % 2])
            copies[c % 2].wait()
            ch = bufs[c % 2]

            def vbody(k, _, _c=c, _ch=ch):
                vb = k * (lanes * unroll)
                for u in range(unroll):
                    off = vb + u * lanes
                    idx = _ch[pl.ds(off, lanes)]
                    loc = idx - lo
                    m = loc.astype(jnp.uint32) < jnp.uint32(vpt)
                    locc = loc & (vpt - 1)
                    jv = iota1 + (_c * chunk + off)
                    plsc.store_scatter(verv, [locc], jv, mask=m)
                return 0

            lax.fori_loop(0, chunk // (lanes * unroll), vbody, 0)

        pltpu.sync_copy(verv, ver_hbm.at[pl.ds(lo, vpt)])

    return ka(put_idx)


def _gather_mem_cols(mem128, sample_idx):
    info = plsc.get_sparse_core_info()
    nc, ns, lanes = info.num_cores, info.num_subcores, info.num_lanes
    nw = nc * ns
    n_sample = sample_idx.shape[0]
    spt = n_sample // nw
    chunk = 256
    nchunk = spt // chunk
    assert spt % chunk == 0 and chunk % lanes == 0

    mesh = plsc.VectorSubcoreMesh(core_axis_name="c", subcore_axis_name="s")

    @functools.partial(
        pl.kernel,
        mesh=mesh,
        out_type=tuple(
            jax.ShapeDtypeStruct((n_sample,), jnp.float32) for _ in range(8)),
        scratch_types=[
            pltpu.VMEM((spt,), jnp.int32),
            pltpu.VMEM((spt,), jnp.int32),
            pltpu.VMEM((chunk, 128), jnp.float32),
            pltpu.VMEM((chunk, 128), jnp.float32),
            pltpu.VMEM((8, spt), jnp.float32),
            pltpu.SemaphoreType.DMA,
            pltpu.SemaphoreType.DMA,
        ],
        compiler_params=pltpu.CompilerParams(needs_layout_passes=False),
    )
    def km(mem_hbm, sidx_hbm, o0, o1, o2, o3, o4, o5, o6, o7,
           sidxv, sg, bb0, bb1, colsv, sem0, sem1):
        wid = lax.axis_index("s") * nc + lax.axis_index("c")
        base = wid * spt
        outs = (o0, o1, o2, o3, o4, o5, o6, o7)
        pltpu.sync_copy(sidx_hbm.at[pl.ds(base, spt)], sidxv)

        def gbody(k, _):
            s = sidxv[pl.ds(k * lanes, lanes)]
            sg[pl.ds(k * lanes, lanes)] = s >> 4
            return 0

        lax.fori_loop(0, spt // lanes, gbody, 0)

        iota = jnp.arange(lanes, dtype=jnp.int32)
        bufs = (bb0, bb1)
        sems = (sem0, sem1)
        copies = [None, None]
        copies[0] = pltpu.async_copy(
            mem_hbm.at[sg.at[pl.ds(0, chunk)]], bb0, sem0)
        for c in range(nchunk):
            if c + 1 < nchunk:
                copies[(c + 1) % 2] = pltpu.async_copy(
                    mem_hbm.at[sg.at[pl.ds((c + 1) * chunk, chunk)]],
                    bufs[(c + 1) % 2], sems[(c + 1) % 2])
            copies[c % 2].wait()
            bb = bufs[c % 2]

            def ebody(k, _, _c=c, _bb=bb):
                off = _c * chunk + k * lanes
                sv = sidxv[pl.ds(off, lanes)]
                sub = (sv & 15) * 8
                rvec = iota + k * lanes
                for co in range(8):
                    val = plsc.load_gather(_bb, [rvec, sub + co])
                    colsv[co, pl.ds(off, lanes)] = val
                return 0

            lax.fori_loop(0, chunk // lanes, ebody, 0)

        for co in range(8):
            pltpu.sync_copy(colsv.at[co], outs[co].at[pl.ds(base, spt)])

    return km(mem128, sample_idx)


def _sample(mem_cols, put_val, sample_idx, ver2d):
    info = plsc.get_sparse_core_info()
    nc, ns, lanes = info.num_cores, info.num_subcores, info.num_lanes
    nw = nc * ns
    n_sample = sample_idx.shape[0]
    spt = n_sample // nw
    assert spt % lanes == 0

    mesh = plsc.VectorSubcoreMesh(core_axis_name="c", subcore_axis_name="s")

    @functools.partial(
        pl.kernel,
        mesh=mesh,
        out_type=tuple(
            jax.ShapeDtypeStruct((n_sample,), jnp.float32) for _ in range(8)),
        scratch_types=[
            pltpu.VMEM((spt,), jnp.int32),
            pltpu.VMEM((spt,), jnp.int32),
            pltpu.VMEM((spt, 8), jnp.int32),
            pltpu.VMEM((spt,), jnp.int32),
            pltpu.VMEM((spt,), jnp.int32),
            pltpu.VMEM((8, spt), jnp.float32),
            pltpu.VMEM((spt, 8), jnp.float32),
            pltpu.VMEM((8, spt), jnp.float32),
            pltpu.SemaphoreType.DMA,
            pltpu.SemaphoreType.DMA,
        ],
        compiler_params=pltpu.CompilerParams(
            needs_layout_passes=False, use_tc_tiling_on_sc=False),
    )
    def kb(m0, m1, m2, m3, m4, m5, m6, m7, pval_hbm, sidx_hbm, ver_hbm,
           o0, o1, o2, o3, o4, o5, o6, o7,
           sidxv, sg, vrows, vv, pv, mcolv, pvalv, selv, sem1, sem2):
        wid = lax.axis_index("s") * nc + lax.axis_index("c")
        base = wid * spt
        outs = (o0, o1, o2, o3, o4, o5, o6, o7)
        mins = (m0, m1, m2, m3, m4, m5, m6, m7)
        pltpu.sync_copy(sidx_hbm.at[pl.ds(base, spt)], sidxv)
        for co in range(8):
            pltpu.sync_copy(mins[co].at[pl.ds(base, spt)], mcolv.at[co])

        def gbody(k, _):
            s = sidxv[pl.ds(k * lanes, lanes)]
            sg[pl.ds(k * lanes, lanes)] = s >> 3
            return 0

        lax.fori_loop(0, spt // lanes, gbody, 0)
        cp_ver = pltpu.async_copy(ver_hbm.at[sg], vrows, sem2)
        cp_ver.wait()

        iota = jnp.arange(lanes, dtype=jnp.int32)

        def pbody(k, _):
            s = sidxv[pl.ds(k * lanes, lanes)]
            rvec = iota + k * lanes
            v = plsc.load_gather(vrows, [rvec, s & 7])
            vv[pl.ds(k * lanes, lanes)] = v
            pv[pl.ds(k * lanes, lanes)] = jnp.where(v > 0, v - 1, base + rvec)
            return 0

        lax.fori_loop(0, spt // lanes, pbody, 0)
        cp_pval = pltpu.async_copy(pval_hbm.at[pv], pvalv, sem2)
        cp_pval.wait()

        def sbody(k, _):
            vvv = vv[pl.ds(k * lanes, lanes)]
            m = vvv > 0
            rvec = iota + k * lanes
            for co in range(8):
                cosplat = jnp.full((lanes,), co, jnp.int32)
                mval = mcolv[co, pl.ds(k * lanes, lanes)]
                pval = plsc.load_gather(pvalv, [rvec, cosplat])
                sel = jnp.where(m, pval, mval)
                selv[co, pl.ds(k * lanes, lanes)] = sel
            return 0

        lax.fori_loop(0, spt // lanes, sbody, 0)

        for co in range(8):
            pltpu.sync_copy(selv.at[co], outs[co].at[pl.ds(base, spt)])

    return kb(*mem_cols, put_val, sample_idx, ver2d)


def kernel(mem, put_idx, put_val, sample_idx):
    put_idx = put_idx.astype(jnp.int32)
    sample_idx = sample_idx.astype(jnp.int32)
    ver = _build_ver(put_idx, mem.shape[0])
    ver2d = ver.reshape(-1, 8)
    mem128 = mem.reshape(-1, 128)
    mem_cols = _gather_mem_cols(mem128, sample_idx)
    cols = _sample(mem_cols, put_val, sample_idx, ver2d)
    state = jnp.stack(cols[0:3], axis=1)
    action = cols[3].reshape(-1, 1)
    reward = cols[4].reshape(-1, 1)
    next_state = jnp.stack(cols[5:8], axis=1)
    return (state, action, reward, next_state)


# speedup vs baseline: 1.4979x; 1.1482x over previous
import functools

import jax
import jax.numpy as jnp
from jax import lax
from jax.experimental import pallas as pl
from jax.experimental.pallas import tpu as pltpu
from jax.experimental.pallas import tpu_sc as plsc


def _build_ver(put_idx, max_size):
    info = plsc.get_sparse_core_info()
    nc, ns, lanes = info.num_cores, info.num_subcores, info.num_lanes
    nw = nc * ns
    n_put = put_idx.shape[0]
    vpt = 1 << max(-(-max_size // nw) - 1, 1).bit_length()
    ver_total = vpt * nw
    chunk = 16384
    nchunk = n_put // chunk
    unroll = 4
    assert n_put % chunk == 0 and chunk % (lanes * unroll) == 0

    mesh = plsc.VectorSubcoreMesh(core_axis_name="c", subcore_axis_name="s")

    @functools.partial(
        pl.kernel,
        mesh=mesh,
        out_type=jax.ShapeDtypeStruct((ver_total,), jnp.int32),
        scratch_types=[
            pltpu.VMEM((chunk,), jnp.int32),
            pltpu.VMEM((chunk,), jnp.int32),
            pltpu.VMEM((vpt,), jnp.int32),
            pltpu.SemaphoreType.DMA,
            pltpu.SemaphoreType.DMA,
        ],
        compiler_params=pltpu.CompilerParams(needs_layout_passes=False),
    )
    def ka(put_hbm, ver_hbm, ch0, ch1, verv, sem0, sem1):
        wid = lax.axis_index("s") * nc + lax.axis_index("c")
        lo = wid * vpt
        zero16 = jnp.zeros((lanes,), jnp.int32)
        iota1 = jnp.arange(lanes, dtype=jnp.int32) + 1

        bufs = (ch0, ch1)
        sems = (sem0, sem1)
        copies = [None, None]
        copies[0] = pltpu.async_copy(put_hbm.at[pl.ds(0, chunk)], ch0, sem0)

        def zbody(i, _):
            zb = i * (lanes * 8)
            for u in range(8):
                verv[pl.ds(zb + u * lanes, lanes)] = zero16
            return 0

        lax.fori_loop(0, vpt // (lanes * 8), zbody, 0)

        for c in range(nchunk):
            if c + 1 < nchunk:
                copies[(c + 1) % 2] = pltpu.async_copy(
                    put_hbm.at[pl.ds((c + 1) * chunk, chunk)],
                    bufs[(c + 1) % 2], sems[(c + 1) % 2])
            copies[c % 2].wait()
            ch = bufs[c % 2]

            def vbody(k, _, _c=c, _ch=ch):
                vb = k * (lanes * unroll)
                for u in range(unroll):
                    off = vb + u * lanes
                    idx = _ch[pl.ds(off, lanes)]
                    loc = idx - lo
                    m = loc.astype(jnp.uint32) < jnp.uint32(vpt)
                    locc = loc & (vpt - 1)
                    jv = iota1 + (_c * chunk + off)
                    plsc.store_scatter(verv, [locc], jv, mask=m)
                return 0

            lax.fori_loop(0, chunk // (lanes * unroll), vbody, 0)

        pltpu.sync_copy(verv, ver_hbm.at[pl.ds(lo, vpt)])

    return ka(put_idx)


def _repack(mem):
    n = mem.shape[0]
    mem3d = mem.reshape(n // 16, 16, 8)
    blk = 1000
    grid = (n // 16) // blk
    assert blk * grid * 16 == n

    def body(in_ref, out_ref):
        for q in range(16):
            out_ref[:, q * 8:(q + 1) * 8] = in_ref[:, q, :]

    return pl.pallas_call(
        body,
        grid=(grid,),
        in_specs=[pl.BlockSpec((blk, 16, 8), lambda i: (i, 0, 0))],
        out_specs=pl.BlockSpec((blk, 128), lambda i: (i, 0)),
        out_shape=jax.ShapeDtypeStruct((n // 16, 128), jnp.float32),
    )(mem3d)


def _gather_mem_cols(mem128, sample_idx):
    info = plsc.get_sparse_core_info()
    nc, ns, lanes = info.num_cores, info.num_subcores, info.num_lanes
    nw = nc * ns
    n_sample = sample_idx.shape[0]
    spt = n_sample // nw
    chunk = 256
    nchunk = spt // chunk
    assert spt % chunk == 0 and chunk % lanes == 0

    mesh = plsc.VectorSubcoreMesh(core_axis_name="c", subcore_axis_name="s")

    @functools.partial(
        pl.kernel,
        mesh=mesh,
        out_type=tuple(
            jax.ShapeDtypeStruct((n_sample,), jnp.float32) for _ in range(8)),
        scratch_types=[
            pltpu.VMEM((spt,), jnp.int32),
            pltpu.VMEM((spt,), jnp.int32),
            pltpu.VMEM((chunk, 128), jnp.float32),
            pltpu.VMEM((chunk, 128), jnp.float32),
            pltpu.VMEM((8, spt), jnp.float32),
            pltpu.SemaphoreType.DMA,
            pltpu.SemaphoreType.DMA,
        ],
        compiler_params=pltpu.CompilerParams(needs_layout_passes=False),
    )
    def km(mem_hbm, sidx_hbm, o0, o1, o2, o3, o4, o5, o6, o7,
           sidxv, sg, bb0, bb1, colsv, sem0, sem1):
        wid = lax.axis_index("s") * nc + lax.axis_index("c")
        base = wid * spt
        outs = (o0, o1, o2, o3, o4, o5, o6, o7)
        pltpu.sync_copy(sidx_hbm.at[pl.ds(base, spt)], sidxv)

        def gbody(k, _):
            s = sidxv[pl.ds(k * lanes, lanes)]
            sg[pl.ds(k * lanes, lanes)] = s >> 4
            return 0

        lax.fori_loop(0, spt // lanes, gbody, 0)

        iota = jnp.arange(lanes, dtype=jnp.int32)
        bufs = (bb0, bb1)
        sems = (sem0, sem1)
        copies = [None, None]
        copies[0] = pltpu.async_copy(
            mem_hbm.at[sg.at[pl.ds(0, chunk)]], bb0, sem0)
        for c in range(nchunk):
            if c + 1 < nchunk:
                copies[(c + 1) % 2] = pltpu.async_copy(
                    mem_hbm.at[sg.at[pl.ds((c + 1) * chunk, chunk)]],
                    bufs[(c + 1) % 2], sems[(c + 1) % 2])
            copies[c % 2].wait()
            bb = bufs[c % 2]

            def ebody(k, _, _c=c, _bb=bb):
                off = _c * chunk + k * lanes
                sv = sidxv[pl.ds(off, lanes)]
                sub = (sv & 15) * 8
                rvec = iota + k * lanes
                for co in range(8):
                    val = plsc.load_gather(_bb, [rvec, sub + co])
                    colsv[co, pl.ds(off, lanes)] = val
                return 0

            lax.fori_loop(0, chunk // lanes, ebody, 0)

        for co in range(8):
            pltpu.sync_copy(colsv.at[co], outs[co].at[pl.ds(base, spt)])

    return km(mem128, sample_idx)


def _sample(mem_cols, put_val, sample_idx, ver2d):
    info = plsc.get_sparse_core_info()
    nc, ns, lanes = info.num_cores, info.num_subcores, info.num_lanes
    nw = nc * ns
    n_sample = sample_idx.shape[0]
    spt = n_sample // nw
    assert spt % lanes == 0

    mesh = plsc.VectorSubcoreMesh(core_axis_name="c", subcore_axis_name="s")

    @functools.partial(
        pl.kernel,
        mesh=mesh,
        out_type=tuple(
            jax.ShapeDtypeStruct((n_sample,), jnp.float32) for _ in range(8)),
        scratch_types=[
            pltpu.VMEM((spt,), jnp.int32),
            pltpu.VMEM((spt,), jnp.int32),
            pltpu.VMEM((spt, 8), jnp.int32),
            pltpu.VMEM((spt,), jnp.int32),
            pltpu.VMEM((spt,), jnp.int32),
            pltpu.VMEM((8, spt), jnp.float32),
            pltpu.VMEM((spt, 8), jnp.float32),
            pltpu.VMEM((8, spt), jnp.float32),
            pltpu.SemaphoreType.DMA,
            pltpu.SemaphoreType.DMA,
        ],
        compiler_params=pltpu.CompilerParams(
            needs_layout_passes=False, use_tc_tiling_on_sc=False),
    )
    def kb(m0, m1, m2, m3, m4, m5, m6, m7, pval_hbm, sidx_hbm, ver_hbm,
           o0, o1, o2, o3, o4, o5, o6, o7,
           sidxv, sg, vrows, vv, pv, mcolv, pvalv, selv, sem1, sem2):
        wid = lax.axis_index("s") * nc + lax.axis_index("c")
        base = wid * spt
        outs = (o0, o1, o2, o3, o4, o5, o6, o7)
        mins = (m0, m1, m2, m3, m4, m5, m6, m7)
        pltpu.sync_copy(sidx_hbm.at[pl.ds(base, spt)], sidxv)
        for co in range(8):
            pltpu.sync_copy(mins[co].at[pl.ds(base, spt)], mcolv.at[co])

        def gbody(k, _):
            s = sidxv[pl.ds(k * lanes, lanes)]
            sg[pl.ds(k * lanes, lanes)] = s >> 3
            return 0

        lax.fori_loop(0, spt // lanes, gbody, 0)
        cp_ver = pltpu.async_copy(ver_hbm.at[sg], vrows, sem2)
        cp_ver.wait()

        iota = jnp.arange(lanes, dtype=jnp.int32)

        def pbody(k, _):
            s = sidxv[pl.ds(k * lanes, lanes)]
            rvec = iota + k * lanes
            v = plsc.load_gather(vrows, [rvec, s & 7])
            vv[pl.ds(k * lanes, lanes)] = v
            pv[pl.ds(k * lanes, lanes)] = jnp.where(v > 0, v - 1, base + rvec)
            return 0

        lax.fori_loop(0, spt // lanes, pbody, 0)
        cp_pval = pltpu.async_copy(pval_hbm.at[pv], pvalv, sem2)
        cp_pval.wait()

        def sbody(k, _):
            vvv = vv[pl.ds(k * lanes, lanes)]
            m = vvv > 0
            rvec = iota + k * lanes
            for co in range(8):
                cosplat = jnp.full((lanes,), co, jnp.int32)
                mval = mcolv[co, pl.ds(k * lanes, lanes)]
                pval = plsc.load_gather(pvalv, [rvec, cosplat])
                sel = jnp.where(m, pval, mval)
                selv[co, pl.ds(k * lanes, lanes)] = sel
            return 0

        lax.fori_loop(0, spt // lanes, sbody, 0)

        for co in range(8):
            pltpu.sync_copy(selv.at[co], outs[co].at[pl.ds(base, spt)])

    return kb(*mem_cols, put_val, sample_idx, ver2d)


def kernel(mem, put_idx, put_val, sample_idx):
    put_idx = put_idx.astype(jnp.int32)
    sample_idx = sample_idx.astype(jnp.int32)
    ver = _build_ver(put_idx, mem.shape[0])
    ver2d = ver.reshape(-1, 8)
    mem128 = _repack(mem)
    mem_cols = _gather_mem_cols(mem128, sample_idx)
    cols = _sample(mem_cols, put_val, sample_idx, ver2d)
    state = jnp.stack(cols[0:3], axis=1)
    action = cols[3].reshape(-1, 1)
    reward = cols[4].reshape(-1, 1)
    next_state = jnp.stack(cols[5:8], axis=1)
    return (state, action, reward, next_state)
